# Initial kernel scaffold; baseline (speedup 1.0000x reference)
import jax
import jax.numpy as jnp
from jax.experimental import pallas as pl


def kernel(cls_logits_l0, cls_logits_l1, cls_logits_l2, cls_logits_l3, bbox_reg_l0, bbox_reg_l1, bbox_reg_l2, bbox_reg_l3, anchors_l0, anchors_l1, anchors_l2, anchors_l3):
    raise NotImplementedError("write your pallas kernel here")


# speedup vs baseline: 11.0351x; 11.0351x over previous
import functools
import math

import jax
import jax.numpy as jnp
from jax import lax
from jax.experimental import pallas as pl
from jax.experimental.pallas import tpu as pltpu

B = 2
C = 80
IMG = 800.0
SCORE_THRESH = 0.05
NMS_THRESH = 0.5
DETS = 100
BBOX_CLAMP = 4.135166556742356
K_ANC = 128
T_LOGIT = -math.log((1.0 - SCORE_THRESH) / SCORE_THRESH)


def _amax_body(n, a_blk, x_ref, o_ref):
    i = pl.program_id(1)
    x = x_ref[0]
    xm = jnp.max(jnp.where(x > T_LOGIT, x, -1e30), axis=1)
    m = jnp.where(xm > -1e29, 1.0 / (1.0 + jnp.exp(-xm)), -1.0)
    rows = i * a_blk + lax.broadcasted_iota(jnp.int32, (a_blk,), 0)
    m = jnp.where(rows < n, m, -2.0)
    o_ref[0, 0] = m


def _anchor_max(x, a_blk):
    n = x.shape[1]
    nb = pl.cdiv(n, a_blk)
    out = pl.pallas_call(
        functools.partial(_amax_body, n, a_blk),
        grid=(B, nb),
        in_specs=[pl.BlockSpec((1, a_blk, C), lambda b, i: (b, i, 0))],
        out_specs=pl.BlockSpec((1, 1, a_blk), lambda b, i: (b * nb + i, 0, 0)),
        out_shape=jax.ShapeDtypeStruct((B * nb, 1, a_blk), jnp.float32),
    )(x)
    return out.reshape(B, nb * a_blk)


def _decode(rel, anc):
    w = anc[:, 2] - anc[:, 0]
    h = anc[:, 3] - anc[:, 1]
    cx = anc[:, 0] + 0.5 * w
    cy = anc[:, 1] + 0.5 * h
    dx, dy = rel[:, 0], rel[:, 1]
    dw = jnp.minimum(rel[:, 2], BBOX_CLAMP)
    dh = jnp.minimum(rel[:, 3], BBOX_CLAMP)
    pcx = dx * w + cx
    pcy = dy * h + cy
    pw = jnp.exp(dw) * w
    ph = jnp.exp(dh) * h
    return jnp.stack([pcx - 0.5 * pw, pcy - 0.5 * ph,
                      pcx + 0.5 * pw, pcy + 0.5 * ph], axis=1)


def _iou_one(b, boxes):
    x1 = jnp.maximum(b[0], boxes[:, 0])
    y1 = jnp.maximum(b[1], boxes[:, 1])
    x2 = jnp.minimum(b[2], boxes[:, 2])
    y2 = jnp.minimum(b[3], boxes[:, 3])
    inter = jnp.maximum(x2 - x1, 0.0) * jnp.maximum(y2 - y1, 0.0)
    ab = (b[2] - b[0]) * (b[3] - b[1])
    ar = (boxes[:, 2] - boxes[:, 0]) * (boxes[:, 3] - boxes[:, 1])
    return inter / (ab + ar - inter + 1e-9)


def kernel(cls_logits_l0, cls_logits_l1, cls_logits_l2, cls_logits_l3,
           bbox_reg_l0, bbox_reg_l1, bbox_reg_l2, bbox_reg_l3,
           anchors_l0, anchors_l1, anchors_l2, anchors_l3):
    logits = [cls_logits_l0, cls_logits_l1, cls_logits_l2, cls_logits_l3]
    regs = [bbox_reg_l0, bbox_reg_l1, bbox_reg_l2, bbox_reg_l3]
    ancs = [anchors_l0, anchors_l1, anchors_l2, anchors_l3]
    blks = [2048, 2048, 2048, 1536]

    m_list = [_anchor_max(logits[l], blks[l]) for l in range(4)]

    outs = []
    for b in range(B):
        all_b, all_s, all_l = [], [], []
        for l in range(4):
            _, a_sel = lax.top_k(m_list[l][b], K_ANC)
            a_sel = jnp.sort(a_sel)
            glog = logits[l][b][a_sel]
            s = jax.nn.sigmoid(glog)
            sc = jnp.where(s > SCORE_THRESH, s, -1.0).reshape(-1)
            top_s, top_i = lax.top_k(sc, DETS)
            a_idx = a_sel[top_i // C]
            labels = top_i % C
            boxes = _decode(regs[l][b][a_idx], ancs[l][a_idx])
            boxes = jnp.clip(boxes, 0.0, IMG)
            all_b.append(boxes)
            all_s.append(top_s)
            all_l.append(labels)
        boxes = jnp.concatenate(all_b, axis=0)
        scores = jnp.concatenate(all_s, axis=0)
        labels = jnp.concatenate(all_l, axis=0)
        offs = labels.astype(jnp.float32) * (IMG + 1.0)
        nb = boxes + offs[:, None]

        def step(work, _):
            j = jnp.argmax(work)
            ious = _iou_one(nb[j], nb)
            work = jnp.where(ious > NMS_THRESH, -jnp.inf, work)
            work = work.at[j].set(-jnp.inf)
            return work, j

        _, keep = lax.scan(step, scores, None, length=DETS)
        fs = scores[keep]
        valid = fs > SCORE_THRESH
        out_b = jnp.where(valid[:, None], boxes[keep], 0.0)
        out_s = jnp.where(valid, fs, 0.0)
        out_l = jnp.where(valid, labels[keep], 0)
        outs.append((out_b, out_s, out_l))
    return (jnp.stack([o[0] for o in outs], axis=0),
            jnp.stack([o[1] for o in outs], axis=0),
            jnp.stack([o[2] for o in outs], axis=0))


# speedup vs baseline: 32.2097x; 2.9188x over previous
import functools
import math

import jax
import jax.numpy as jnp
from jax import lax
from jax.experimental import pallas as pl
from jax.experimental.pallas import tpu as pltpu

B = 2
C = 80
IMG = 800.0
SCORE_THRESH = 0.05
NMS_THRESH = 0.5
DETS = 100
BBOX_CLAMP = 4.135166556742356
K_ANC = 128
T_LOGIT = -math.log((1.0 - SCORE_THRESH) / SCORE_THRESH)


def _amax_body(n, a_blk, x_ref, o_ref):
    i = pl.program_id(1)
    x = x_ref[0]
    xm = jnp.max(jnp.where(x > T_LOGIT, x, -1e30), axis=1)
    m = jnp.where(xm > -1e29, 1.0 / (1.0 + jnp.exp(-xm)), -1.0)
    rows = i * a_blk + lax.broadcasted_iota(jnp.int32, (a_blk,), 0)
    m = jnp.where(rows < n, m, -2.0)
    o_ref[0, 0] = m


def _anchor_max(x, a_blk):
    n = x.shape[1]
    nb = pl.cdiv(n, a_blk)
    out = pl.pallas_call(
        functools.partial(_amax_body, n, a_blk),
        grid=(B, nb),
        in_specs=[pl.BlockSpec((1, a_blk, C), lambda b, i: (b, i, 0))],
        out_specs=pl.BlockSpec((1, 1, a_blk), lambda b, i: (b * nb + i, 0, 0)),
        out_shape=jax.ShapeDtypeStruct((B * nb, 1, a_blk), jnp.float32),
    )(x)
    return out.reshape(B, nb * a_blk)


def _decode(rel, anc):
    w = anc[:, 2] - anc[:, 0]
    h = anc[:, 3] - anc[:, 1]
    cx = anc[:, 0] + 0.5 * w
    cy = anc[:, 1] + 0.5 * h
    dx, dy = rel[:, 0], rel[:, 1]
    dw = jnp.minimum(rel[:, 2], BBOX_CLAMP)
    dh = jnp.minimum(rel[:, 3], BBOX_CLAMP)
    pcx = dx * w + cx
    pcy = dy * h + cy
    pw = jnp.exp(dw) * w
    ph = jnp.exp(dh) * h
    return jnp.stack([pcx - 0.5 * pw, pcy - 0.5 * ph,
                      pcx + 0.5 * pw, pcy + 0.5 * ph], axis=1)


def _nms_body(bx_ref, sc_ref, lb_ref, ob_ref, os_ref, ol_ref):
    ncand = sc_ref.shape[2]
    x1 = bx_ref[0, 0:1, :]
    y1 = bx_ref[0, 1:2, :]
    x2 = bx_ref[0, 2:3, :]
    y2 = bx_ref[0, 3:4, :]
    scores = sc_ref[0]
    labf = lb_ref[0].astype(jnp.float32)
    offs = labf * (IMG + 1.0)
    nx1, ny1, nx2, ny2 = x1 + offs, y1 + offs, x2 + offs, y2 + offs
    area = (nx2 - nx1) * (ny2 - ny1)
    iota = lax.broadcasted_iota(jnp.int32, (1, ncand), 1)
    kiota = lax.broadcasted_iota(jnp.int32, (1, DETS), 1)
    zrow = jnp.zeros((1, DETS), jnp.float32)

    def step(i, carry):
        work, fs, fl, b1, b2, b3, b4 = carry
        mx = jnp.max(work)
        j = jnp.min(jnp.where(work == mx, iota, ncand))
        jm = iota == j

        def ext(row):
            return jnp.sum(jnp.where(jm, row, 0.0))

        jx1, jy1, jx2, jy2 = ext(nx1), ext(ny1), ext(nx2), ext(ny2)
        ja = (jx2 - jx1) * (jy2 - jy1)
        inter = (jnp.maximum(jnp.minimum(jx2, nx2) - jnp.maximum(jx1, nx1), 0.0)
                 * jnp.maximum(jnp.minimum(jy2, ny2) - jnp.maximum(jy1, ny1), 0.0))
        iou = inter / (ja + area - inter + 1e-9)
        im = kiota == i
        fs = jnp.where(im, ext(scores), fs)
        fl = jnp.where(im, ext(labf), fl)
        b1 = jnp.where(im, ext(x1), b1)
        b2 = jnp.where(im, ext(y1), b2)
        b3 = jnp.where(im, ext(x2), b3)
        b4 = jnp.where(im, ext(y2), b4)
        work = jnp.where(iou > NMS_THRESH, -jnp.inf, work)
        work = jnp.where(jm, -jnp.inf, work)
        return work, fs, fl, b1, b2, b3, b4

    _, fs, fl, b1, b2, b3, b4 = lax.fori_loop(
        0, DETS, step, (scores, zrow, zrow, zrow, zrow, zrow, zrow))

    valid = fs > SCORE_THRESH
    vf = valid.astype(jnp.float32)
    os_ref[0] = jnp.where(valid, fs, 0.0)
    ol_ref[0] = jnp.where(valid, fl, 0.0).astype(jnp.int32)
    ob_ref[0, 0:1, :] = b1 * vf
    ob_ref[0, 1:2, :] = b2 * vf
    ob_ref[0, 2:3, :] = b3 * vf
    ob_ref[0, 3:4, :] = b4 * vf


def _nms(boxes_t, scores, labels):
    ncand = scores.shape[1]
    ob, os_, ol = pl.pallas_call(
        _nms_body,
        grid=(B,),
        in_specs=[
            pl.BlockSpec((1, 4, ncand), lambda b: (b, 0, 0)),
            pl.BlockSpec((1, 1, ncand), lambda b: (b, 0, 0)),
            pl.BlockSpec((1, 1, ncand), lambda b: (b, 0, 0)),
        ],
        out_specs=[
            pl.BlockSpec((1, 4, DETS), lambda b: (b, 0, 0)),
            pl.BlockSpec((1, 1, DETS), lambda b: (b, 0, 0)),
            pl.BlockSpec((1, 1, DETS), lambda b: (b, 0, 0)),
        ],
        out_shape=[
            jax.ShapeDtypeStruct((B, 4, DETS), jnp.float32),
            jax.ShapeDtypeStruct((B, 1, DETS), jnp.float32),
            jax.ShapeDtypeStruct((B, 1, DETS), jnp.int32),
        ],
    )(boxes_t, scores[:, None, :], labels[:, None, :])
    return ob, os_[:, 0, :], ol[:, 0, :]


def kernel(cls_logits_l0, cls_logits_l1, cls_logits_l2, cls_logits_l3,
           bbox_reg_l0, bbox_reg_l1, bbox_reg_l2, bbox_reg_l3,
           anchors_l0, anchors_l1, anchors_l2, anchors_l3):
    logits = [cls_logits_l0, cls_logits_l1, cls_logits_l2, cls_logits_l3]
    regs = [bbox_reg_l0, bbox_reg_l1, bbox_reg_l2, bbox_reg_l3]
    ancs = [anchors_l0, anchors_l1, anchors_l2, anchors_l3]
    blks = [2048, 2048, 2048, 1536]

    m_list = [_anchor_max(logits[l], blks[l]) for l in range(4)]

    outs = []
    for b in range(B):
        all_b, all_s, all_l = [], [], []
        for l in range(4):
            _, a_sel = lax.top_k(m_list[l][b], K_ANC)
            a_sel = jnp.sort(a_sel)
            glog = logits[l][b][a_sel]
            s = jax.nn.sigmoid(glog)
            sc = jnp.where(s > SCORE_THRESH, s, -1.0).reshape(-1)
            top_s, top_i = lax.top_k(sc, DETS)
            a_idx = a_sel[top_i // C]
            labels = top_i % C
            boxes = _decode(regs[l][b][a_idx], ancs[l][a_idx])
            boxes = jnp.clip(boxes, 0.0, IMG)
            all_b.append(boxes)
            all_s.append(top_s)
            all_l.append(labels)
        outs.append((jnp.concatenate(all_b, axis=0),
                     jnp.concatenate(all_s, axis=0),
                     jnp.concatenate(all_l, axis=0)))
    boxes_t = jnp.stack([o[0].T for o in outs], axis=0)
    scores = jnp.stack([o[1] for o in outs], axis=0)
    labels = jnp.stack([o[2] for o in outs], axis=0)
    ob, os_, ol = _nms(boxes_t, scores, labels)
    return jnp.swapaxes(ob, 1, 2), os_, ol
